# baseline (speedup 1.0000x reference)
import functools

import jax
import jax.numpy as jnp
from jax import lax
from jax.experimental import pallas as pl
from jax.experimental.pallas import tpu as pltpu
from jax.experimental.pallas import tpu_sc as plsc

B, NL, NH, CL, CH, OUT = 8, 4096, 1024, 128, 256, 256
P = B * NL
TQ = 256
TN = 512

NC, NS, LN = 2, 16, 16
NW = NC * NS
PPW = P // NW
CHK = 64
NCHUNK = PPW // CHK
NSEG = CH // LN


def _knn_body(qT_ref, hT_ref, gidx_ref, wt_ref):
    q = qT_ref[0]
    h = hT_ref[0]
    xy = lax.dot_general(q, h, (((0,), (0,)), ((), ())),
                         preferred_element_type=jnp.float32)
    x2 = jnp.sum(q * q, axis=0)
    y2 = jnp.sum(h * h, axis=0)
    d2 = x2[:, None] + y2[None, :] - 2.0 * xy

    iota = lax.broadcasted_iota(jnp.int32, (TQ, NH), 1)
    big = jnp.float32(jnp.inf)

    def min3(d):
        m = jnp.min(d, axis=1)
        cand = jnp.where(d == m[:, None], iota, NH)
        i = jnp.min(cand, axis=1)
        d_next = jnp.where(iota == i[:, None], big, d)
        return m, i, d_next

    m1, i1, d2b = min3(d2)
    m2, i2, d2c = min3(d2b)
    m3, i3, _ = min3(d2c)

    def invd(m):
        d = jnp.sqrt(jnp.maximum(m, 0.0))
        return 1.0 / jnp.maximum(d, 1e-8)

    w1 = invd(m1)
    w2 = invd(m2)
    w3 = invd(m3)
    s = w1 + w2 + w3
    base = pl.program_id(0) * NH
    gidx_ref[0] = jnp.stack([i1 + base, i2 + base, i3 + base], axis=0)
    wt_ref[0] = jnp.stack([w1 / s, w2 / s, w3 / s], axis=0)


def _gather_body(table_ref, gidx_ref, wt_ref, fint_ref,
                 idx_v, w_v, rows_v, out_v, sem):
    wid = lax.axis_index("s") * NC + lax.axis_index("c")
    seg_iota = lax.broadcasted_iota(jnp.int32, (LN,), 0)

    def chunk_body(c, carry):
        cbase = wid * PPW + c * CHK
        for k in range(3):
            pltpu.sync_copy(gidx_ref.at[k, pl.ds(cbase, CHK)],
                            idx_v.at[pl.ds(k * CHK, CHK)])
            pltpu.sync_copy(wt_ref.at[k, pl.ds(cbase, CHK)],
                            w_v.at[pl.ds(k * CHK, CHK)])
        pltpu.async_copy(table_ref.at[idx_v], rows_v, sem).wait()

        def point_body(p, carry2):
            wb = [plsc.load_gather(w_v, [jnp.full((LN,), k * CHK, jnp.int32) + p])
                  for k in range(3)]
            rr = [jnp.full((LN,), k * CHK, jnp.int32) + p for k in range(3)]
            for s in range(NSEG):
                cols = seg_iota + (s * LN)
                acc = wb[0] * plsc.load_gather(rows_v, [rr[0], cols])
                acc += wb[1] * plsc.load_gather(rows_v, [rr[1], cols])
                acc += wb[2] * plsc.load_gather(rows_v, [rr[2], cols])
                plsc.store_scatter(out_v, [jnp.full((LN,), p, jnp.int32), cols],
                                   acc)
            return carry2

        lax.fori_loop(0, CHK, point_body, 0)
        pltpu.sync_copy(out_v, fint_ref.at[pl.ds(cbase, CHK), :])
        return carry

    lax.fori_loop(0, NCHUNK, chunk_body, 0)


def _linear_body(fint_ref, fl_ref, w1_ref, w2_ref, b_ref, y_ref, stats_ref):
    y = lax.dot_general(w1_ref[...], fint_ref[...], (((1,), (1,)), ((), ())),
                        preferred_element_type=jnp.float32)
    y += lax.dot_general(w2_ref[...], fl_ref[0], (((1,), (0,)), ((), ())),
                         preferred_element_type=jnp.float32)
    y += b_ref[...]
    y_ref[0] = y
    part = jnp.stack([jnp.sum(y, axis=1), jnp.sum(y * y, axis=1)], axis=0)
    first = jnp.logical_and(pl.program_id(0) == 0, pl.program_id(1) == 0)

    @pl.when(first)
    def _():
        stats_ref[...] = part

    @pl.when(jnp.logical_not(first))
    def _():
        stats_ref[...] += part


def _norm_body(y_ref, sc_ref, sh_ref, o_ref):
    o_ref[0] = jnp.maximum(y_ref[0] * sc_ref[...] + sh_ref[...], 0.0)


@jax.jit
def kernel(xyz_low, xyz_high, feat_low, feat_high, W, b, gamma, beta):
    qT = jnp.transpose(xyz_low, (0, 2, 1))
    hT = jnp.transpose(xyz_high, (0, 2, 1))
    table = jnp.transpose(feat_high, (0, 2, 1)).reshape(B * NH, CH)
    W1 = W[:, :CH]
    W2 = W[:, CH:]
    bb = b[:, None]

    grid = (B, NL // TQ)
    gidx, wt = pl.pallas_call(
        _knn_body,
        grid=grid,
        in_specs=[
            pl.BlockSpec((1, 3, TQ), lambda bi, i: (bi, 0, i)),
            pl.BlockSpec((1, 3, NH), lambda bi, i: (bi, 0, 0)),
        ],
        out_specs=[
            pl.BlockSpec((1, 3, TQ), lambda bi, i: (bi, 0, i)),
            pl.BlockSpec((1, 3, TQ), lambda bi, i: (bi, 0, i)),
        ],
        out_shape=[
            jax.ShapeDtypeStruct((B, 3, NL), jnp.int32),
            jax.ShapeDtypeStruct((B, 3, NL), jnp.float32),
        ],
    )(qT, hT)
    gidx2 = gidx.transpose(1, 0, 2).reshape(3, P)
    wt2 = wt.transpose(1, 0, 2).reshape(3, P)

    sc_gather = pl.kernel(
        _gather_body,
        out_type=jax.ShapeDtypeStruct((P, CH), jnp.float32),
        mesh=plsc.VectorSubcoreMesh(core_axis_name="c", subcore_axis_name="s"),
        compiler_params=pltpu.CompilerParams(needs_layout_passes=False),
        scratch_types=[
            pltpu.VMEM((3 * CHK,), jnp.int32),
            pltpu.VMEM((3 * CHK,), jnp.float32),
            pltpu.VMEM((3 * CHK, CH), jnp.float32),
            pltpu.VMEM((CHK, CH), jnp.float32),
            pltpu.SemaphoreType.DMA,
        ],
    )
    fint = sc_gather(table, gidx2, wt2)

    y, stats = pl.pallas_call(
        _linear_body,
        grid=grid,
        in_specs=[
            pl.BlockSpec((TQ, CH), lambda bi, i: (bi * (NL // TQ) + i, 0)),
            pl.BlockSpec((1, CL, TQ), lambda bi, i: (bi, 0, i)),
            pl.BlockSpec((OUT, CH), lambda bi, i: (0, 0)),
            pl.BlockSpec((OUT, CL), lambda bi, i: (0, 0)),
            pl.BlockSpec((OUT, 1), lambda bi, i: (0, 0)),
        ],
        out_specs=[
            pl.BlockSpec((1, OUT, TQ), lambda bi, i: (bi, 0, i)),
            pl.BlockSpec((2, OUT), lambda bi, i: (0, 0)),
        ],
        out_shape=[
            jax.ShapeDtypeStruct((B, OUT, NL), jnp.float32),
            jax.ShapeDtypeStruct((2, OUT), jnp.float32),
        ],
    )(fint, feat_low, W1, W2, bb)

    n = jnp.float32(P)
    mean = stats[0] / n
    var = jnp.maximum(stats[1] / n - mean * mean, 0.0)
    scale = gamma / jnp.sqrt(var + 1e-5)
    shift = beta - mean * scale

    out = pl.pallas_call(
        _norm_body,
        grid=(B, NL // TN),
        in_specs=[
            pl.BlockSpec((1, OUT, TN), lambda bi, i: (bi, 0, i)),
            pl.BlockSpec((OUT, 1), lambda bi, i: (0, 0)),
            pl.BlockSpec((OUT, 1), lambda bi, i: (0, 0)),
        ],
        out_specs=pl.BlockSpec((1, OUT, TN), lambda bi, i: (bi, 0, i)),
        out_shape=jax.ShapeDtypeStruct((B, OUT, NL), jnp.float32),
    )(y, scale[:, None], shift[:, None])
    return out


# speedup vs baseline: 1.1028x; 1.1028x over previous
import functools

import jax
import jax.numpy as jnp
from jax import lax
from jax.experimental import pallas as pl
from jax.experimental.pallas import tpu as pltpu
from jax.experimental.pallas import tpu_sc as plsc

B, NL, NH, CL, CH, OUT = 8, 4096, 1024, 128, 256, 256
P = B * NL
TQ = 256
TN = 512

NC, NS, LN = 2, 16, 16
NW = NC * NS
PPW = P // NW
CHK = 64
NCHUNK = PPW // CHK
NSEG = CH // LN


def _knn_body(qT_ref, hT_ref, gidx_ref, wt_ref):
    q = qT_ref[0]
    h = hT_ref[0]
    xym2 = lax.dot_general(-2.0 * q, h, (((0,), (0,)), ((), ())),
                           preferred_element_type=jnp.float32)
    x2 = jnp.sum(q * q, axis=0)
    y2 = jnp.sum(h * h, axis=0)
    r2 = y2[None, :] + xym2

    iota = lax.broadcasted_iota(jnp.int32, (TQ, NH), 1).astype(jnp.float32)
    big = jnp.float32(jnp.inf)
    nhf = jnp.float32(NH)

    def min3(d):
        m = jnp.min(d, axis=1)
        cand = jnp.where(d == m[:, None], iota, nhf)
        i = jnp.min(cand, axis=1)
        d_next = jnp.where(cand == i[:, None], big, d)
        return m, i, d_next

    m1, i1, r2b = min3(r2)
    m2, i2, r2c = min3(r2b)
    m3, i3, _ = min3(r2c)

    def invd(m):
        d = jnp.sqrt(jnp.maximum(m + x2, 0.0))
        return 1.0 / jnp.maximum(d, 1e-8)

    w1 = invd(m1)
    w2 = invd(m2)
    w3 = invd(m3)
    s = w1 + w2 + w3
    base = pl.program_id(0) * NH
    gidx_ref[0] = jnp.stack([i1.astype(jnp.int32) + base,
                             i2.astype(jnp.int32) + base,
                             i3.astype(jnp.int32) + base], axis=0)
    wt_ref[0] = jnp.stack([w1 / s, w2 / s, w3 / s], axis=0)


def _gather_body(table_ref, gidx_ref, wt_ref, fint_ref,
                 i00, i01, i02, i10, i11, i12, w_v,
                 r00, r01, r02, r10, r11, r12, out_v, sem0, sem1):
    wid = lax.axis_index("s") * NC + lax.axis_index("c")
    wbase = wid * PPW
    seg_iota = lax.broadcasted_iota(jnp.int32, (LN,), 0)
    idxb = ((i00, i01, i02), (i10, i11, i12))
    rows = ((r00, r01, r02), (r10, r11, r12))
    sems = (sem0, sem1)

    for k in range(3):
        pltpu.sync_copy(wt_ref.at[pl.ds(k * P + wbase, PPW)],
                        w_v.at[pl.ds(k * PPW, PPW)])

    def fire(c):
        par = c % 2
        hs = []
        for k in range(3):
            pltpu.sync_copy(gidx_ref.at[pl.ds(k * P + wbase + c * CHK, CHK)],
                            idxb[par][k])
            hs.append(pltpu.async_copy(table_ref.at[idxb[par][k]],
                                       rows[par][k], sems[par]))
        return hs

    pending = {0: fire(0)}
    for c in range(NCHUNK):
        if c + 1 < NCHUNK:
            pending[c + 1] = fire(c + 1)
        for h in pending.pop(c):
            h.wait()
        par = c % 2

        def point_body(p, carry2, _par=par, _c=c):
            wb = [plsc.load_gather(
                      w_v, [jnp.full((LN,), k * PPW + _c * CHK, jnp.int32) + p])
                  for k in range(3)]
            pidx = jnp.full((LN,), 0, jnp.int32) + p
            for s in range(NSEG):
                cols = seg_iota + (s * LN)
                acc = wb[0] * plsc.load_gather(rows[_par][0], [pidx, cols])
                acc += wb[1] * plsc.load_gather(rows[_par][1], [pidx, cols])
                acc += wb[2] * plsc.load_gather(rows[_par][2], [pidx, cols])
                plsc.store_scatter(out_v, [pidx, cols], acc)
            return carry2

        lax.fori_loop(0, CHK, point_body, 0)
        pltpu.sync_copy(out_v, fint_ref.at[pl.ds(wbase + c * CHK, CHK), :])


def _linear_body(fint_ref, fl_ref, w1_ref, w2_ref, b_ref, y_ref, stats_ref):
    y = lax.dot_general(w1_ref[...], fint_ref[...], (((1,), (1,)), ((), ())),
                        preferred_element_type=jnp.float32)
    y += lax.dot_general(w2_ref[...], fl_ref[0], (((1,), (0,)), ((), ())),
                         preferred_element_type=jnp.float32)
    y += b_ref[...]
    y_ref[0] = y
    part = jnp.stack([jnp.sum(y, axis=1), jnp.sum(y * y, axis=1)], axis=0)
    first = jnp.logical_and(pl.program_id(0) == 0, pl.program_id(1) == 0)

    @pl.when(first)
    def _():
        stats_ref[...] = part

    @pl.when(jnp.logical_not(first))
    def _():
        stats_ref[...] += part


def _norm_body(y_ref, sc_ref, sh_ref, o_ref):
    o_ref[0] = jnp.maximum(y_ref[0] * sc_ref[...] + sh_ref[...], 0.0)


@jax.jit
def kernel(xyz_low, xyz_high, feat_low, feat_high, W, b, gamma, beta):
    qT = jnp.transpose(xyz_low, (0, 2, 1))
    hT = jnp.transpose(xyz_high, (0, 2, 1))
    table = jnp.transpose(feat_high, (0, 2, 1)).reshape(B * NH, CH)
    W1 = W[:, :CH]
    W2 = W[:, CH:]
    bb = b[:, None]

    grid = (B, NL // TQ)
    gidx, wt = pl.pallas_call(
        _knn_body,
        grid=grid,
        in_specs=[
            pl.BlockSpec((1, 3, TQ), lambda bi, i: (bi, 0, i)),
            pl.BlockSpec((1, 3, NH), lambda bi, i: (bi, 0, 0)),
        ],
        out_specs=[
            pl.BlockSpec((1, 3, TQ), lambda bi, i: (bi, 0, i)),
            pl.BlockSpec((1, 3, TQ), lambda bi, i: (bi, 0, i)),
        ],
        out_shape=[
            jax.ShapeDtypeStruct((B, 3, NL), jnp.int32),
            jax.ShapeDtypeStruct((B, 3, NL), jnp.float32),
        ],
    )(qT, hT)
    gidx2 = gidx.transpose(1, 0, 2).reshape(3 * P)
    wt2 = wt.transpose(1, 0, 2).reshape(3 * P)

    sc_gather = pl.kernel(
        _gather_body,
        out_type=jax.ShapeDtypeStruct((P, CH), jnp.float32),
        mesh=plsc.VectorSubcoreMesh(core_axis_name="c", subcore_axis_name="s"),
        compiler_params=pltpu.CompilerParams(needs_layout_passes=False),
        scratch_types=[
            pltpu.VMEM((CHK,), jnp.int32),
            pltpu.VMEM((CHK,), jnp.int32),
            pltpu.VMEM((CHK,), jnp.int32),
            pltpu.VMEM((CHK,), jnp.int32),
            pltpu.VMEM((CHK,), jnp.int32),
            pltpu.VMEM((CHK,), jnp.int32),
            pltpu.VMEM((3 * PPW,), jnp.float32),
            pltpu.VMEM((CHK, CH), jnp.float32),
            pltpu.VMEM((CHK, CH), jnp.float32),
            pltpu.VMEM((CHK, CH), jnp.float32),
            pltpu.VMEM((CHK, CH), jnp.float32),
            pltpu.VMEM((CHK, CH), jnp.float32),
            pltpu.VMEM((CHK, CH), jnp.float32),
            pltpu.VMEM((CHK, CH), jnp.float32),
            pltpu.SemaphoreType.DMA,
            pltpu.SemaphoreType.DMA,
        ],
    )
    fint = sc_gather(table, gidx2, wt2)

    y, stats = pl.pallas_call(
        _linear_body,
        grid=grid,
        in_specs=[
            pl.BlockSpec((TQ, CH), lambda bi, i: (bi * (NL // TQ) + i, 0)),
            pl.BlockSpec((1, CL, TQ), lambda bi, i: (bi, 0, i)),
            pl.BlockSpec((OUT, CH), lambda bi, i: (0, 0)),
            pl.BlockSpec((OUT, CL), lambda bi, i: (0, 0)),
            pl.BlockSpec((OUT, 1), lambda bi, i: (0, 0)),
        ],
        out_specs=[
            pl.BlockSpec((1, OUT, TQ), lambda bi, i: (bi, 0, i)),
            pl.BlockSpec((2, OUT), lambda bi, i: (0, 0)),
        ],
        out_shape=[
            jax.ShapeDtypeStruct((B, OUT, NL), jnp.float32),
            jax.ShapeDtypeStruct((2, OUT), jnp.float32),
        ],
    )(fint, feat_low, W1, W2, bb)

    n = jnp.float32(P)
    mean = stats[0] / n
    var = jnp.maximum(stats[1] / n - mean * mean, 0.0)
    scale = gamma / jnp.sqrt(var + 1e-5)
    shift = beta - mean * scale

    out = pl.pallas_call(
        _norm_body,
        grid=(B, NL // TN),
        in_specs=[
            pl.BlockSpec((1, OUT, TN), lambda bi, i: (bi, 0, i)),
            pl.BlockSpec((OUT, 1), lambda bi, i: (0, 0)),
            pl.BlockSpec((OUT, 1), lambda bi, i: (0, 0)),
        ],
        out_specs=pl.BlockSpec((1, OUT, TN), lambda bi, i: (bi, 0, i)),
        out_shape=jax.ShapeDtypeStruct((B, OUT, NL), jnp.float32),
    )(y, scale[:, None], shift[:, None])
    return out
